# baseline (speedup 1.0000x reference)
import functools

import jax
import jax.numpy as jnp
from jax import lax
from jax.experimental import pallas as pl
from jax.experimental.pallas import tpu as pltpu
from jax.experimental.pallas import tpu_sc as plsc

N = 10000
E = 320000
C = 128
D = 16
G = 64
EPS = 1e-5

NC = 2
NS = 16
NW = NC * NS
EW = E // NW
EWP = 10240
K = 32
NCHUNK = EWP // K
NPAD = 10112
ZR = NPAD // NS
EPAD = 322560
EPBLK = 2520


def _node_proj_body(x_ref, wa_ref, wb_ref, a_ref, b_ref):
    xv = x_ref[...]
    a_ref[...] = jnp.dot(xv, wa_ref[...], preferred_element_type=jnp.float32)
    b_ref[...] = jnp.dot(xv, wb_ref[...], preferred_element_type=jnp.float32)


def _node_proj(x_pad, wa, wb):
    blk = 1264
    grid = (NPAD // blk,)
    return pl.pallas_call(
        _node_proj_body,
        grid=grid,
        in_specs=[
            pl.BlockSpec((blk, C), lambda i: (i, 0)),
            pl.BlockSpec((C, 2 * C), lambda i: (0, 0)),
            pl.BlockSpec((C, 2 * C), lambda i: (0, 0)),
        ],
        out_specs=[
            pl.BlockSpec((blk, 2 * C), lambda i: (i, 0)),
            pl.BlockSpec((blk, 2 * C), lambda i: (i, 0)),
        ],
        out_shape=[
            jax.ShapeDtypeStruct((NPAD, 2 * C), jnp.float32),
            jax.ShapeDtypeStruct((NPAD, 2 * C), jnp.float32),
        ],
    )(x_pad, wa, wb)


def _edge_proj_body(ea_ref, w0_ref, b0_ref, w1_ref, b1_ref, ep0_ref, ep1_ref):
    ea = ea_ref[...]
    ep0_ref[...] = jnp.dot(ea, w0_ref[...], preferred_element_type=jnp.float32) + b0_ref[...]
    ep1_ref[...] = jnp.dot(ea, w1_ref[...], preferred_element_type=jnp.float32) + b1_ref[...]


def _edge_proj(ea_pad, w0, b0, w1, b1):
    grid = (EPAD // EPBLK,)
    return pl.pallas_call(
        _edge_proj_body,
        grid=grid,
        in_specs=[
            pl.BlockSpec((EPBLK, D), lambda i: (i, 0)),
            pl.BlockSpec((D, 2 * C), lambda i: (0, 0)),
            pl.BlockSpec((1, 2 * C), lambda i: (0, 0)),
            pl.BlockSpec((D, 2 * C), lambda i: (0, 0)),
            pl.BlockSpec((1, 2 * C), lambda i: (0, 0)),
        ],
        out_specs=[
            pl.BlockSpec((EPBLK, 2 * C), lambda i: (i, 0)),
            pl.BlockSpec((EPBLK, 2 * C), lambda i: (i, 0)),
        ],
        out_shape=[
            jax.ShapeDtypeStruct((EPAD, 2 * C), jnp.float32),
            jax.ShapeDtypeStruct((EPAD, 2 * C), jnp.float32),
        ],
    )(ea_pad, w0, b0, w1, b1)


def _sc_edge_body(src_hbm, dst_hbm, a_hbm, b_hbm, ep_hbm, zeros_hbm,
                  out_hbm,
                  shared,
                  src0, src1, dst0, dst1, ra0, ra1, rb0, rb1, ep_v, m0, m1,
                  sg0, sg1, ss0, ss1):
    c = lax.axis_index("c")
    s = lax.axis_index("s")
    wid = c * NS + s
    base = wid * EWP
    ebase = wid * EW

    srcs = (src0, src1)
    dsts = (dst0, dst1)
    ras = (ra0, ra1)
    rbs = (rb0, rb1)
    ms = (m0, m1)
    sgs = (sg0, sg1)
    sss = (ss0, ss1)

    pltpu.sync_copy(zeros_hbm.at[pl.ds(s * ZR, ZR)],
                    shared.at[pl.ds(s * ZR, ZR)])
    plsc.subcore_barrier()

    def fetch(nb, cn):
        eb = base + cn * K
        pltpu.sync_copy(src_hbm.at[pl.ds(eb, K)], srcs[nb])
        pltpu.sync_copy(dst_hbm.at[pl.ds(eb, K)], dsts[nb])
        pltpu.async_copy(a_hbm.at[dsts[nb]], ras[nb], sgs[nb])
        pltpu.async_copy(b_hbm.at[srcs[nb]], rbs[nb], sgs[nb])

    def wait_gathers(b):
        pltpu.make_async_copy(a_hbm.at[dsts[b]], ras[b], sgs[b]).wait()
        pltpu.make_async_copy(b_hbm.at[srcs[b]], rbs[b], sgs[b]).wait()

    def wait_scatter(b):
        pltpu.make_async_copy(ms[b], shared.at[dsts[b]], sss[b]).wait()

    def compute(b):
        ra = ras[b]
        rb = rbs[b]
        mb = ms[b]

        def edge(i, _):
            for j in range(C // 16):
                q = 16 * j
                f = ra[i, pl.ds(q, 16)] + rb[i, pl.ds(q, 16)] + ep_v[i, pl.ds(q, 16)]
                sv = ra[i, pl.ds(C + q, 16)] + rb[i, pl.ds(C + q, 16)] + ep_v[i, pl.ds(C + q, 16)]
                sig = 1.0 / (1.0 + jnp.exp(-f))
                t = jnp.exp(-jnp.abs(sv))
                u = t / (2.0 + t)
                u2 = u * u
                l1p = 2.0 * u * (1.0 + u2 * (1.0 / 3.0 + u2 * (0.2 + u2 * (1.0 / 7.0 + u2 * (1.0 / 9.0)))))
                sp = jnp.maximum(sv, 0.0) + l1p
                mb[i, pl.ds(q, 16)] = sig * sp
            return 0

        lax.fori_loop(0, K, edge, 0)

    fetch(0, 0)

    def super_chunk(g, _):
        for b in range(2):
            ci = 2 * g + b
            nb = 1 - b

            wait_gathers(b)

            @pl.when(ci >= 1)
            def _():
                wait_scatter(nb)

            @pl.when(ci + 1 < NCHUNK)
            def _():
                fetch(nb, ci + 1)

            pltpu.sync_copy(ep_hbm.at[pl.ds(ebase + ci * K, K)], ep_v)
            compute(b)
            pltpu.async_copy(ms[b], shared.at[dsts[b]], sss[b], add=True)
        return 0

    lax.fori_loop(0, NCHUNK // 2, super_chunk, 0)
    wait_scatter(1)

    plsc.subcore_barrier()
    pltpu.sync_copy(shared.at[pl.ds(s * ZR, ZR)],
                    out_hbm.at[c, pl.ds(s * ZR, ZR)])


def _sc_edge_pass(src_pad, dst_pad, a, b, ep, zeros_n):
    kern = pl.kernel(
        _sc_edge_body,
        out_type=jax.ShapeDtypeStruct((NC, NPAD, C), jnp.float32),
        mesh=plsc.VectorSubcoreMesh(core_axis_name="c", subcore_axis_name="s"),
        scratch_types=[
            pltpu.VMEM_SHARED((NPAD, C), jnp.float32),
            pltpu.VMEM((K,), jnp.int32),
            pltpu.VMEM((K,), jnp.int32),
            pltpu.VMEM((K,), jnp.int32),
            pltpu.VMEM((K,), jnp.int32),
            pltpu.VMEM((K, 2 * C), jnp.float32),
            pltpu.VMEM((K, 2 * C), jnp.float32),
            pltpu.VMEM((K, 2 * C), jnp.float32),
            pltpu.VMEM((K, 2 * C), jnp.float32),
            pltpu.VMEM((K, 2 * C), jnp.float32),
            pltpu.VMEM((K, C), jnp.float32),
            pltpu.VMEM((K, C), jnp.float32),
            pltpu.SemaphoreType.DMA,
            pltpu.SemaphoreType.DMA,
            pltpu.SemaphoreType.DMA,
            pltpu.SemaphoreType.DMA,
        ],
    )
    return kern(src_pad, dst_pad, a, b, ep, zeros_n)


def _bn(h, g_row, be_row):
    mu = jnp.mean(h, axis=0, keepdims=True)
    hc = h - mu
    var = jnp.mean(hc * hc, axis=0, keepdims=True)
    return hc * lax.rsqrt(var + EPS) * g_row + be_row


def _mid_body(x_ref, p_ref, g_ref, be_ref, wa_ref, wb_ref,
              hn_ref, a_ref, b_ref):
    h = x_ref[...] + p_ref[0] + p_ref[1]
    hn = _bn(h, g_ref[...], be_ref[...])
    hn_ref[...] = hn
    pad = jnp.zeros((NPAD - N, 2 * C), jnp.float32)
    a = jnp.dot(hn, wa_ref[...], preferred_element_type=jnp.float32)
    b = jnp.dot(hn, wb_ref[...], preferred_element_type=jnp.float32)
    a_ref[...] = jnp.concatenate([a, pad])
    b_ref[...] = jnp.concatenate([b, pad])


def _mid(x, p, g, be, wa, wb):
    return pl.pallas_call(
        _mid_body,
        grid=(1,),
        in_specs=[
            pl.BlockSpec((N, C), lambda i: (0, 0)),
            pl.BlockSpec((NC, N, C), lambda i: (0, 0, 0)),
            pl.BlockSpec((1, C), lambda i: (0, 0)),
            pl.BlockSpec((1, C), lambda i: (0, 0)),
            pl.BlockSpec((C, 2 * C), lambda i: (0, 0)),
            pl.BlockSpec((C, 2 * C), lambda i: (0, 0)),
        ],
        out_specs=[
            pl.BlockSpec((N, C), lambda i: (0, 0)),
            pl.BlockSpec((NPAD, 2 * C), lambda i: (0, 0)),
            pl.BlockSpec((NPAD, 2 * C), lambda i: (0, 0)),
        ],
        out_shape=[
            jax.ShapeDtypeStruct((N, C), jnp.float32),
            jax.ShapeDtypeStruct((NPAD, 2 * C), jnp.float32),
            jax.ShapeDtypeStruct((NPAD, 2 * C), jnp.float32),
        ],
    )(x, p, g.reshape(1, C), be.reshape(1, C), wa, wb)


def _final_body(h1_ref, p_ref, g_ref, be_ref, batch_ref,
                w1_ref, b1_ref, w2_ref, b2_ref, out_ref):
    h = h1_ref[...] + p_ref[0] + p_ref[1]
    hn = _bn(h, g_ref[...], be_ref[...])
    bi = batch_ref[...]
    gid = lax.broadcasted_iota(jnp.int32, (1, G), 1)
    onehot = (bi == gid).astype(jnp.float32)
    sums = lax.dot_general(onehot, hn, (((0,), (0,)), ((), ())),
                           preferred_element_type=jnp.float32)
    counts = jnp.sum(onehot, axis=0)[:, None]
    pooled = sums / jnp.maximum(counts, 1.0)
    z = jnp.dot(pooled, w1_ref[...], preferred_element_type=jnp.float32) + b1_ref[...]
    z = jnp.where(z > 0, z, 0.01 * z)
    out_ref[...] = jnp.dot(z, w2_ref[...], preferred_element_type=jnp.float32) + b2_ref[...]


def _final(h1, p, g, be, batch, w1, b1, w2, b2):
    return pl.pallas_call(
        _final_body,
        grid=(1,),
        in_specs=[
            pl.BlockSpec((N, C), lambda i: (0, 0)),
            pl.BlockSpec((NC, N, C), lambda i: (0, 0, 0)),
            pl.BlockSpec((1, C), lambda i: (0, 0)),
            pl.BlockSpec((1, C), lambda i: (0, 0)),
            pl.BlockSpec((N, 1), lambda i: (0, 0)),
            pl.BlockSpec((C, C), lambda i: (0, 0)),
            pl.BlockSpec((1, C), lambda i: (0, 0)),
            pl.BlockSpec((C, C), lambda i: (0, 0)),
            pl.BlockSpec((1, C), lambda i: (0, 0)),
        ],
        out_specs=pl.BlockSpec((G, C), lambda i: (0, 0)),
        out_shape=jax.ShapeDtypeStruct((G, C), jnp.float32),
    )(h1, p, g.reshape(1, C), be.reshape(1, C),
      batch.reshape(N, 1), w1, b1.reshape(1, C), w2, b2.reshape(1, C))


def kernel(x, edge_index, edge_attr, batch, Wf0, bf0, Ws0, bs0, g0, be0,
           Wf1, bf1, Ws1, bs1, g1, be1, W1, b1, W2, b2):
    src_pad = jnp.pad(edge_index[0].reshape(NW, EW),
                      ((0, 0), (0, EWP - EW))).reshape(NW * EWP)
    dst_pad = jnp.pad(edge_index[1].reshape(NW, EW),
                      ((0, 0), (0, EWP - EW)),
                      constant_values=NPAD - 1).reshape(NW * EWP)

    wa0 = jnp.concatenate([Wf0[:C], Ws0[:C]], axis=1)
    wb0 = jnp.concatenate([Wf0[C:2 * C], Ws0[C:2 * C]], axis=1)
    we0 = jnp.concatenate([Wf0[2 * C:], Ws0[2 * C:]], axis=1)
    eb0 = jnp.concatenate([bf0, bs0]).reshape(1, 2 * C)
    wa1 = jnp.concatenate([Wf1[:C], Ws1[:C]], axis=1)
    wb1 = jnp.concatenate([Wf1[C:2 * C], Ws1[C:2 * C]], axis=1)
    we1 = jnp.concatenate([Wf1[2 * C:], Ws1[2 * C:]], axis=1)
    eb1 = jnp.concatenate([bf1, bs1]).reshape(1, 2 * C)

    x_pad = jnp.pad(x, ((0, NPAD - N), (0, 0)))
    ea_pad = jnp.pad(edge_attr, ((0, EPAD - E), (0, 0)))
    zeros_n = jnp.zeros((NPAD, C), jnp.float32)

    a0, b0 = _node_proj(x_pad, wa0, wb0)
    ep0, ep1 = _edge_proj(ea_pad, we0, eb0, we1, eb1)

    p_l0 = _sc_edge_pass(src_pad, dst_pad, a0, b0, ep0, zeros_n)
    h1, a1, b1_ = _mid(x, p_l0, g0, be0, wa1, wb1)

    p_l1 = _sc_edge_pass(src_pad, dst_pad, a1, b1_, ep1, zeros_n)
    return _final(h1, p_l1, g1, be1, batch, W1, b1, W2, b2)


# speedup vs baseline: 3.6127x; 3.6127x over previous
import functools

import jax
import jax.numpy as jnp
from jax import lax
from jax.experimental import pallas as pl
from jax.experimental.pallas import tpu as pltpu
from jax.experimental.pallas import tpu_sc as plsc

N = 10000
E = 320000
C = 128
D = 16
G = 64
EPS = 1e-5

NC = 2
NS = 16
NW = NC * NS
EW = E // NW
EWP = 10240
K = 32
NCHUNK = EWP // K
NPAD = 10112
ZR = NPAD // NS
EPAD = 322560
EPBLK = 2520


def _node_proj_body(x_ref, wa_ref, wb_ref, a_ref, b_ref):
    xv = x_ref[...]
    a_ref[...] = jnp.dot(xv, wa_ref[...], preferred_element_type=jnp.float32)
    b_ref[...] = jnp.dot(xv, wb_ref[...], preferred_element_type=jnp.float32)


def _node_proj(x_pad, wa, wb):
    blk = 1264
    grid = (NPAD // blk,)
    return pl.pallas_call(
        _node_proj_body,
        grid=grid,
        in_specs=[
            pl.BlockSpec((blk, C), lambda i: (i, 0)),
            pl.BlockSpec((C, 2 * C), lambda i: (0, 0)),
            pl.BlockSpec((C, 2 * C), lambda i: (0, 0)),
        ],
        out_specs=[
            pl.BlockSpec((blk, 2 * C), lambda i: (i, 0)),
            pl.BlockSpec((blk, 2 * C), lambda i: (i, 0)),
        ],
        out_shape=[
            jax.ShapeDtypeStruct((NPAD, 2 * C), jnp.float32),
            jax.ShapeDtypeStruct((NPAD, 2 * C), jnp.float32),
        ],
    )(x_pad, wa, wb)


def _edge_proj_body(ea_ref, w0_ref, b0_ref, w1_ref, b1_ref, ep0_ref, ep1_ref):
    ea = ea_ref[...]
    ep0_ref[...] = jnp.dot(ea, w0_ref[...], preferred_element_type=jnp.float32) + b0_ref[...]
    ep1_ref[...] = jnp.dot(ea, w1_ref[...], preferred_element_type=jnp.float32) + b1_ref[...]


def _edge_proj(ea_pad, w0, b0, w1, b1):
    grid = (EPAD // EPBLK,)
    return pl.pallas_call(
        _edge_proj_body,
        grid=grid,
        in_specs=[
            pl.BlockSpec((EPBLK, D), lambda i: (i, 0)),
            pl.BlockSpec((D, 2 * C), lambda i: (0, 0)),
            pl.BlockSpec((1, 2 * C), lambda i: (0, 0)),
            pl.BlockSpec((D, 2 * C), lambda i: (0, 0)),
            pl.BlockSpec((1, 2 * C), lambda i: (0, 0)),
        ],
        out_specs=[
            pl.BlockSpec((EPBLK, 2 * C), lambda i: (i, 0)),
            pl.BlockSpec((EPBLK, 2 * C), lambda i: (i, 0)),
        ],
        out_shape=[
            jax.ShapeDtypeStruct((EPAD, 2 * C), jnp.float32),
            jax.ShapeDtypeStruct((EPAD, 2 * C), jnp.float32),
        ],
    )(ea_pad, w0, b0, w1, b1)


def _sc_edge_body(src_hbm, dst_hbm, a_hbm, b_hbm, ep_hbm, zeros_hbm,
                  out_hbm,
                  shared,
                  src0, src1, dst0, dst1, ra0, ra1, rb0, rb1, ep_v, m0, m1,
                  sg0, sg1, ss0, ss1):
    c = lax.axis_index("c")
    s = lax.axis_index("s")
    wid = c * NS + s
    base = wid * EWP
    ebase = wid * EW

    srcs = (src0, src1)
    dsts = (dst0, dst1)
    ras = (ra0, ra1)
    rbs = (rb0, rb1)
    ms = (m0, m1)
    sgs = (sg0, sg1)
    sss = (ss0, ss1)

    pltpu.sync_copy(zeros_hbm.at[pl.ds(s * ZR, ZR)],
                    shared.at[pl.ds(s * ZR, ZR)])
    plsc.subcore_barrier()

    def fetch(nb, cn):
        eb = base + cn * K
        pltpu.sync_copy(src_hbm.at[pl.ds(eb, K)], srcs[nb])
        pltpu.sync_copy(dst_hbm.at[pl.ds(eb, K)], dsts[nb])
        pltpu.async_copy(a_hbm.at[dsts[nb]], ras[nb], sgs[nb])
        pltpu.async_copy(b_hbm.at[srcs[nb]], rbs[nb], sgs[nb])

    def wait_gathers(b):
        pltpu.make_async_copy(a_hbm.at[dsts[b]], ras[b], sgs[b]).wait()
        pltpu.make_async_copy(b_hbm.at[srcs[b]], rbs[b], sgs[b]).wait()

    def wait_scatter(b):
        pltpu.make_async_copy(ms[b], shared.at[dsts[b]], sss[b]).wait()

    def compute(b):
        ra = ras[b]
        rb = rbs[b]
        mb = ms[b]

        @functools.partial(plsc.parallel_loop, 0, K, unroll=4)
        def edge(i):
            for j in range(C // 16):
                q = 16 * j
                f = ra[i, pl.ds(q, 16)] + rb[i, pl.ds(q, 16)] + ep_v[i, pl.ds(q, 16)]
                sv = ra[i, pl.ds(C + q, 16)] + rb[i, pl.ds(C + q, 16)] + ep_v[i, pl.ds(C + q, 16)]
                sig = 1.0 / (1.0 + jnp.exp(-f))
                t = jnp.exp(-jnp.abs(sv))
                u = t / (2.0 + t)
                u2 = u * u
                l1p = 2.0 * u * (1.0 + u2 * (1.0 / 3.0 + u2 * (0.2 + u2 * (1.0 / 7.0 + u2 * (1.0 / 9.0)))))
                sp = jnp.maximum(sv, 0.0) + l1p
                mb[i, pl.ds(q, 16)] = sig * sp

    fetch(0, 0)

    def super_chunk(g, _):
        for b in range(2):
            ci = 2 * g + b
            nb = 1 - b

            wait_gathers(b)

            @pl.when(ci >= 1)
            def _():
                wait_scatter(nb)

            @pl.when(ci + 1 < NCHUNK)
            def _():
                fetch(nb, ci + 1)

            pltpu.sync_copy(ep_hbm.at[pl.ds(ebase + ci * K, K)], ep_v)
            compute(b)
            pltpu.async_copy(ms[b], shared.at[dsts[b]], sss[b], add=True)
        return 0

    lax.fori_loop(0, NCHUNK // 2, super_chunk, 0)
    wait_scatter(1)

    plsc.subcore_barrier()
    pltpu.sync_copy(shared.at[pl.ds(s * ZR, ZR)],
                    out_hbm.at[c, pl.ds(s * ZR, ZR)])


def _sc_edge_pass(src_pad, dst_pad, a, b, ep, zeros_n):
    kern = pl.kernel(
        _sc_edge_body,
        out_type=jax.ShapeDtypeStruct((NC, NPAD, C), jnp.float32),
        mesh=plsc.VectorSubcoreMesh(core_axis_name="c", subcore_axis_name="s"),
        scratch_types=[
            pltpu.VMEM_SHARED((NPAD, C), jnp.float32),
            pltpu.VMEM((K,), jnp.int32),
            pltpu.VMEM((K,), jnp.int32),
            pltpu.VMEM((K,), jnp.int32),
            pltpu.VMEM((K,), jnp.int32),
            pltpu.VMEM((K, 2 * C), jnp.float32),
            pltpu.VMEM((K, 2 * C), jnp.float32),
            pltpu.VMEM((K, 2 * C), jnp.float32),
            pltpu.VMEM((K, 2 * C), jnp.float32),
            pltpu.VMEM((K, 2 * C), jnp.float32),
            pltpu.VMEM((K, C), jnp.float32),
            pltpu.VMEM((K, C), jnp.float32),
            pltpu.SemaphoreType.DMA,
            pltpu.SemaphoreType.DMA,
            pltpu.SemaphoreType.DMA,
            pltpu.SemaphoreType.DMA,
        ],
    )
    return kern(src_pad, dst_pad, a, b, ep, zeros_n)


def _bn(h, g_row, be_row):
    mu = jnp.mean(h, axis=0, keepdims=True)
    hc = h - mu
    var = jnp.mean(hc * hc, axis=0, keepdims=True)
    return hc * lax.rsqrt(var + EPS) * g_row + be_row


def _mid_body(x_ref, p_ref, g_ref, be_ref, wa_ref, wb_ref,
              hn_ref, a_ref, b_ref):
    h = x_ref[...] + p_ref[0] + p_ref[1]
    hn = _bn(h, g_ref[...], be_ref[...])
    hn_ref[...] = hn
    pad = jnp.zeros((NPAD - N, 2 * C), jnp.float32)
    a = jnp.dot(hn, wa_ref[...], preferred_element_type=jnp.float32)
    b = jnp.dot(hn, wb_ref[...], preferred_element_type=jnp.float32)
    a_ref[...] = jnp.concatenate([a, pad])
    b_ref[...] = jnp.concatenate([b, pad])


def _mid(x, p, g, be, wa, wb):
    return pl.pallas_call(
        _mid_body,
        grid=(1,),
        in_specs=[
            pl.BlockSpec((N, C), lambda i: (0, 0)),
            pl.BlockSpec((NC, N, C), lambda i: (0, 0, 0)),
            pl.BlockSpec((1, C), lambda i: (0, 0)),
            pl.BlockSpec((1, C), lambda i: (0, 0)),
            pl.BlockSpec((C, 2 * C), lambda i: (0, 0)),
            pl.BlockSpec((C, 2 * C), lambda i: (0, 0)),
        ],
        out_specs=[
            pl.BlockSpec((N, C), lambda i: (0, 0)),
            pl.BlockSpec((NPAD, 2 * C), lambda i: (0, 0)),
            pl.BlockSpec((NPAD, 2 * C), lambda i: (0, 0)),
        ],
        out_shape=[
            jax.ShapeDtypeStruct((N, C), jnp.float32),
            jax.ShapeDtypeStruct((NPAD, 2 * C), jnp.float32),
            jax.ShapeDtypeStruct((NPAD, 2 * C), jnp.float32),
        ],
    )(x, p, g.reshape(1, C), be.reshape(1, C), wa, wb)


def _final_body(h1_ref, p_ref, g_ref, be_ref, batch_ref,
                w1_ref, b1_ref, w2_ref, b2_ref, out_ref):
    h = h1_ref[...] + p_ref[0] + p_ref[1]
    hn = _bn(h, g_ref[...], be_ref[...])
    bi = batch_ref[...]
    gid = lax.broadcasted_iota(jnp.int32, (1, G), 1)
    onehot = (bi == gid).astype(jnp.float32)
    sums = lax.dot_general(onehot, hn, (((0,), (0,)), ((), ())),
                           preferred_element_type=jnp.float32)
    counts = jnp.sum(onehot, axis=0)[:, None]
    pooled = sums / jnp.maximum(counts, 1.0)
    z = jnp.dot(pooled, w1_ref[...], preferred_element_type=jnp.float32) + b1_ref[...]
    z = jnp.where(z > 0, z, 0.01 * z)
    out_ref[...] = jnp.dot(z, w2_ref[...], preferred_element_type=jnp.float32) + b2_ref[...]


def _final(h1, p, g, be, batch, w1, b1, w2, b2):
    return pl.pallas_call(
        _final_body,
        grid=(1,),
        in_specs=[
            pl.BlockSpec((N, C), lambda i: (0, 0)),
            pl.BlockSpec((NC, N, C), lambda i: (0, 0, 0)),
            pl.BlockSpec((1, C), lambda i: (0, 0)),
            pl.BlockSpec((1, C), lambda i: (0, 0)),
            pl.BlockSpec((N, 1), lambda i: (0, 0)),
            pl.BlockSpec((C, C), lambda i: (0, 0)),
            pl.BlockSpec((1, C), lambda i: (0, 0)),
            pl.BlockSpec((C, C), lambda i: (0, 0)),
            pl.BlockSpec((1, C), lambda i: (0, 0)),
        ],
        out_specs=pl.BlockSpec((G, C), lambda i: (0, 0)),
        out_shape=jax.ShapeDtypeStruct((G, C), jnp.float32),
    )(h1, p, g.reshape(1, C), be.reshape(1, C),
      batch.reshape(N, 1), w1, b1.reshape(1, C), w2, b2.reshape(1, C))


def kernel(x, edge_index, edge_attr, batch, Wf0, bf0, Ws0, bs0, g0, be0,
           Wf1, bf1, Ws1, bs1, g1, be1, W1, b1, W2, b2):
    src_pad = jnp.pad(edge_index[0].reshape(NW, EW),
                      ((0, 0), (0, EWP - EW))).reshape(NW * EWP)
    dst_pad = jnp.pad(edge_index[1].reshape(NW, EW),
                      ((0, 0), (0, EWP - EW)),
                      constant_values=NPAD - 1).reshape(NW * EWP)

    wa0 = jnp.concatenate([Wf0[:C], Ws0[:C]], axis=1)
    wb0 = jnp.concatenate([Wf0[C:2 * C], Ws0[C:2 * C]], axis=1)
    we0 = jnp.concatenate([Wf0[2 * C:], Ws0[2 * C:]], axis=1)
    eb0 = jnp.concatenate([bf0, bs0]).reshape(1, 2 * C)
    wa1 = jnp.concatenate([Wf1[:C], Ws1[:C]], axis=1)
    wb1 = jnp.concatenate([Wf1[C:2 * C], Ws1[C:2 * C]], axis=1)
    we1 = jnp.concatenate([Wf1[2 * C:], Ws1[2 * C:]], axis=1)
    eb1 = jnp.concatenate([bf1, bs1]).reshape(1, 2 * C)

    x_pad = jnp.pad(x, ((0, NPAD - N), (0, 0)))
    ea_pad = jnp.pad(edge_attr, ((0, EPAD - E), (0, 0)))
    zeros_n = jnp.zeros((NPAD, C), jnp.float32)

    a0, b0 = _node_proj(x_pad, wa0, wb0)
    ep0, ep1 = _edge_proj(ea_pad, we0, eb0, we1, eb1)

    p_l0 = _sc_edge_pass(src_pad, dst_pad, a0, b0, ep0, zeros_n)
    h1, a1, b1_ = _mid(x, p_l0, g0, be0, wa1, wb1)

    p_l1 = _sc_edge_pass(src_pad, dst_pad, a1, b1_, ep1, zeros_n)
    return _final(h1, p_l1, g1, be1, batch, W1, b1, W2, b2)
